# baseline (speedup 1.0000x reference)
import functools

import jax
import jax.numpy as jnp
from jax.experimental import pallas as pl

B, F, D = 4096, 26, 128
ROWS = B * F


def _matmul_kernel(idx_ref, x_ref, w_ref, b_ref, o_ref):
    perm = idx_ref[0, :]
    iota = jax.lax.broadcasted_iota(jnp.int32, (D, D), 0)
    onehot = (iota == perm[None, :]).astype(jnp.float32)
    w_p = jax.lax.dot(onehot, w_ref[...],
                      preferred_element_type=jnp.float32)
    o_ref[...] = (
        jax.lax.dot(x_ref[...], w_p, preferred_element_type=jnp.float32)
        + b_ref[0, :][None, :]
    )


@functools.partial(jax.jit, static_argnames=("block_rows",))
def _run(x2d, idx2d, W, b2d, block_rows=2048):
    grid = (ROWS // block_rows,)
    return pl.pallas_call(
        _matmul_kernel,
        grid=grid,
        in_specs=[
            pl.BlockSpec((1, D), lambda i: (0, 0)),
            pl.BlockSpec((block_rows, D), lambda i: (i, 0)),
            pl.BlockSpec((D, D), lambda i: (0, 0)),
            pl.BlockSpec((1, D), lambda i: (0, 0)),
        ],
        out_specs=pl.BlockSpec((block_rows, D), lambda i: (i, 0)),
        out_shape=jax.ShapeDtypeStruct((ROWS, D), jnp.float32),
    )(idx2d, x2d, W, b2d)


def kernel(x, permute_idx, W, b):
    x2d = x.reshape(ROWS, D)
    idx2d = permute_idx.astype(jnp.int32).reshape(1, D)
    b2d = b.reshape(1, D)
    out = _run(x2d, idx2d, W, b2d)
    return out.reshape(B, F, D)


# speedup vs baseline: 1.0930x; 1.0930x over previous
import functools

import jax
import jax.numpy as jnp
from jax.experimental import pallas as pl
from jax.experimental.pallas import tpu as pltpu

B, F, D = 4096, 26, 128
ROWS = B * F


def _matmul_kernel(idx_ref, x_ref, w_ref, b_ref, o_ref):
    perm = idx_ref[0, :]
    iota = jax.lax.broadcasted_iota(jnp.int32, (D, D), 0)
    onehot = (iota == perm[None, :]).astype(jnp.float32)
    w_p = jax.lax.dot(onehot, w_ref[...],
                      preferred_element_type=jnp.float32)
    o_ref[...] = (
        jax.lax.dot(x_ref[...], w_p, preferred_element_type=jnp.float32)
        + b_ref[0, :][None, :]
    )


@functools.partial(jax.jit, static_argnames=("block_rows",))
def _run(x2d, idx2d, W, b2d, block_rows=8192):
    grid = (ROWS // block_rows,)
    return pl.pallas_call(
        _matmul_kernel,
        grid=grid,
        in_specs=[
            pl.BlockSpec((1, D), lambda i: (0, 0)),
            pl.BlockSpec((block_rows, D), lambda i: (i, 0)),
            pl.BlockSpec((D, D), lambda i: (0, 0)),
            pl.BlockSpec((1, D), lambda i: (0, 0)),
        ],
        out_specs=pl.BlockSpec((block_rows, D), lambda i: (i, 0)),
        out_shape=jax.ShapeDtypeStruct((ROWS, D), jnp.float32),
        compiler_params=pltpu.CompilerParams(
            dimension_semantics=("parallel",),
        ),
    )(idx2d, x2d, W, b2d)


def kernel(x, permute_idx, W, b):
    x2d = x.reshape(ROWS, D)
    idx2d = permute_idx.astype(jnp.int32).reshape(1, D)
    b2d = b.reshape(1, D)
    out = _run(x2d, idx2d, W, b2d)
    return out.reshape(B, F, D)


# speedup vs baseline: 1.9181x; 1.7550x over previous
import functools

import jax
import jax.numpy as jnp
from jax.experimental import pallas as pl
from jax.experimental.pallas import tpu as pltpu

B, F, D = 4096, 26, 128


def _matmul_kernel(idx_ref, x_ref, w_ref, b_ref, o_ref):
    perm = idx_ref[0, :]
    iota = jax.lax.broadcasted_iota(jnp.int32, (D, D), 0)
    onehot = (iota == perm[None, :]).astype(jnp.float32)
    w_p = jax.lax.dot(onehot, w_ref[...],
                      preferred_element_type=jnp.float32)
    bb = x_ref.shape[0]
    xf = x_ref[...].reshape(bb * F, D)
    y = jax.lax.dot(xf, w_p, preferred_element_type=jnp.float32)
    o_ref[...] = (y + b_ref[0, :][None, :]).reshape(bb, F, D)


@functools.partial(jax.jit, static_argnames=("block_b",))
def _run(x, idx2d, W, b2d, block_b=256):
    grid = (B // block_b,)
    return pl.pallas_call(
        _matmul_kernel,
        grid=grid,
        in_specs=[
            pl.BlockSpec((1, D), lambda i: (0, 0)),
            pl.BlockSpec((block_b, F, D), lambda i: (i, 0, 0)),
            pl.BlockSpec((D, D), lambda i: (0, 0)),
            pl.BlockSpec((1, D), lambda i: (0, 0)),
        ],
        out_specs=pl.BlockSpec((block_b, F, D), lambda i: (i, 0, 0)),
        out_shape=jax.ShapeDtypeStruct((B, F, D), jnp.float32),
        compiler_params=pltpu.CompilerParams(
            dimension_semantics=("parallel",),
        ),
    )(idx2d, x, W, b2d)


def kernel(x, permute_idx, W, b):
    idx2d = permute_idx.astype(jnp.int32).reshape(1, D)
    b2d = b.reshape(1, D)
    return _run(x, idx2d, W, b2d)


# speedup vs baseline: 1.9620x; 1.0229x over previous
import functools

import jax
import jax.numpy as jnp
from jax.experimental import pallas as pl
from jax.experimental.pallas import tpu as pltpu

B, F, D = 4096, 26, 128


def _matmul_kernel(idx_ref, x_ref, w_ref, b_ref, o_ref):
    perm = idx_ref[0, :]
    iota = jax.lax.broadcasted_iota(jnp.int32, (D, D), 0)
    onehot = (iota == perm[None, :]).astype(jnp.float32)
    w_p = jax.lax.dot(onehot, w_ref[...],
                      preferred_element_type=jnp.float32)
    bb = x_ref.shape[0]
    xf = x_ref[...].reshape(bb * F, D)
    y = jax.lax.dot(xf, w_p, preferred_element_type=jnp.float32)
    o_ref[...] = (y + b_ref[0, :][None, :]).reshape(bb, F, D)


@functools.partial(jax.jit, static_argnames=("block_b",))
def _run(x, idx2d, W, b2d, block_b=512):
    grid = (B // block_b,)
    return pl.pallas_call(
        _matmul_kernel,
        grid=grid,
        in_specs=[
            pl.BlockSpec((1, D), lambda i: (0, 0)),
            pl.BlockSpec((block_b, F, D), lambda i: (i, 0, 0)),
            pl.BlockSpec((D, D), lambda i: (0, 0)),
            pl.BlockSpec((1, D), lambda i: (0, 0)),
        ],
        out_specs=pl.BlockSpec((block_b, F, D), lambda i: (i, 0, 0)),
        out_shape=jax.ShapeDtypeStruct((B, F, D), jnp.float32),
        compiler_params=pltpu.CompilerParams(
            dimension_semantics=("parallel",),
        ),
    )(idx2d, x, W, b2d)


def kernel(x, permute_idx, W, b):
    idx2d = permute_idx.astype(jnp.int32).reshape(1, D)
    b2d = b.reshape(1, D)
    return _run(x, idx2d, W, b2d)


# speedup vs baseline: 2.1652x; 1.1036x over previous
import jax
import jax.numpy as jnp
from jax.experimental import pallas as pl
from jax.experimental.pallas import tpu as pltpu

B, F, D = 4096, 26, 128
FP = 32
CB = 128
K = 8
SUB = 4
SB = CB // SUB
N = B // CB


def _body(idx_ref, x_hbm, w_ref, b_ref, o_hbm,
          xs, ys, wp_ref, in_sems, out_sems):
    i = pl.program_id(0)

    def in_copies(c, slot):
        return [pltpu.make_async_copy(
            x_hbm.at[pl.ds(c * CB + s * SB, SB), :, :],
            xs.at[slot, pl.ds(s * SB, SB), pl.ds(0, F), :],
            in_sems.at[slot],
        ) for s in range(SUB)]

    def out_copies(c, slot):
        return [pltpu.make_async_copy(
            ys.at[slot, pl.ds(s * SB, SB), pl.ds(0, F), :],
            o_hbm.at[pl.ds(c * CB + s * SB, SB), :, :],
            out_sems.at[slot],
        ) for s in range(SUB)]

    def start_all(copies):
        for cp in copies:
            cp.start()

    def wait_all(copies):
        for cp in copies:
            cp.wait()

    @pl.when(i == 0)
    def _prologue():
        perm = idx_ref[0, :]
        iota = jax.lax.broadcasted_iota(jnp.int32, (D, D), 0)
        onehot = (iota == perm[None, :]).astype(jnp.float32)
        wp_ref[...] = jax.lax.dot(onehot, w_ref[...],
                                  preferred_element_type=jnp.float32)
        for k in range(K):
            start_all(in_copies(k, k))

    slot = jax.lax.rem(i, K)
    wait_all(in_copies(i, slot))

    @pl.when(i >= K)
    def _():
        wait_all(out_copies(i - K, slot))

    xv = xs[slot]
    y = jax.lax.dot(xv.reshape(CB * FP, D), wp_ref[...],
                    preferred_element_type=jnp.float32)
    y = y + b_ref[0, :][None, :]
    ys[slot] = y.reshape(CB, FP, D)
    start_all(out_copies(i, slot))

    @pl.when(i + K < N)
    def _():
        start_all(in_copies(i + K, slot))

    @pl.when(i == N - 1)
    def _drain():
        for d in range(K):
            c = N - K + d
            if c >= 0:
                wait_all(out_copies(c, c % K))


@jax.jit
def _run(x, idx2d, W, b2d):
    return pl.pallas_call(
        _body,
        grid=(N,),
        in_specs=[
            pl.BlockSpec((1, D), lambda i: (0, 0)),
            pl.BlockSpec(memory_space=pltpu.MemorySpace.HBM),
            pl.BlockSpec((D, D), lambda i: (0, 0)),
            pl.BlockSpec((1, D), lambda i: (0, 0)),
        ],
        out_specs=pl.BlockSpec(memory_space=pltpu.MemorySpace.HBM),
        out_shape=jax.ShapeDtypeStruct((B, F, D), jnp.float32),
        scratch_shapes=[
            pltpu.VMEM((K, CB, FP, D), jnp.float32),
            pltpu.VMEM((K, CB, FP, D), jnp.float32),
            pltpu.VMEM((D, D), jnp.float32),
            pltpu.SemaphoreType.DMA((K,)),
            pltpu.SemaphoreType.DMA((K,)),
        ],
        compiler_params=pltpu.CompilerParams(
            dimension_semantics=("arbitrary",),
        ),
    )(idx2d, x, W, b2d)


def kernel(x, permute_idx, W, b):
    idx2d = permute_idx.astype(jnp.int32).reshape(1, D)
    b2d = b.reshape(1, D)
    return _run(x, idx2d, W, b2d)


# speedup vs baseline: 8.2648x; 3.8171x over previous
import jax
import jax.numpy as jnp
from jax.experimental import pallas as pl
from jax.experimental.pallas import tpu as pltpu

B, F, D = 4096, 26, 128
ROWS = B * F
RB = 4096
K = 8
N = ROWS // RB


def _body(idx_ref, x_hbm, w_ref, b_ref, o_hbm,
          xs, ys, wp_ref, in_sems, out_sems):
    i = pl.program_id(0)

    def in_copy(c, slot):
        return pltpu.make_async_copy(
            x_hbm.at[pl.ds(c * RB, RB), :],
            xs.at[slot],
            in_sems.at[slot],
        )

    def out_copy(c, slot):
        return pltpu.make_async_copy(
            ys.at[slot],
            o_hbm.at[pl.ds(c * RB, RB), :],
            out_sems.at[slot],
        )

    @pl.when(i == 0)
    def _prologue():
        perm = idx_ref[0, :]
        iota = jax.lax.broadcasted_iota(jnp.int32, (D, D), 0)
        onehot = (iota == perm[None, :]).astype(jnp.float32)
        wp_ref[...] = jax.lax.dot(onehot, w_ref[...],
                                  preferred_element_type=jnp.float32)
        for k in range(K):
            in_copy(k, k).start()

    slot = jax.lax.rem(i, K)
    in_copy(i, slot).wait()

    @pl.when(i >= K)
    def _():
        out_copy(i - K, slot).wait()

    y = jax.lax.dot(xs[slot], wp_ref[...],
                    preferred_element_type=jnp.float32)
    ys[slot] = y + b_ref[0, :][None, :]
    out_copy(i, slot).start()

    @pl.when(i + K < N)
    def _():
        in_copy(i + K, slot).start()

    @pl.when(i == N - 1)
    def _drain():
        for d in range(K):
            c = N - K + d
            if c >= 0:
                out_copy(c, c % K).wait()


@jax.jit
def _run(x2d, idx2d, W, b2d):
    return pl.pallas_call(
        _body,
        grid=(N,),
        in_specs=[
            pl.BlockSpec((1, D), lambda i: (0, 0)),
            pl.BlockSpec(memory_space=pltpu.MemorySpace.HBM),
            pl.BlockSpec((D, D), lambda i: (0, 0)),
            pl.BlockSpec((1, D), lambda i: (0, 0)),
        ],
        out_specs=pl.BlockSpec(memory_space=pltpu.MemorySpace.HBM),
        out_shape=jax.ShapeDtypeStruct((ROWS, D), jnp.float32),
        scratch_shapes=[
            pltpu.VMEM((K, RB, D), jnp.float32),
            pltpu.VMEM((K, RB, D), jnp.float32),
            pltpu.VMEM((D, D), jnp.float32),
            pltpu.SemaphoreType.DMA((K,)),
            pltpu.SemaphoreType.DMA((K,)),
        ],
        compiler_params=pltpu.CompilerParams(
            dimension_semantics=("arbitrary",),
        ),
    )(idx2d, x2d, W, b2d)


def kernel(x, permute_idx, W, b):
    idx2d = permute_idx.astype(jnp.int32).reshape(1, D)
    b2d = b.reshape(1, D)
    x2d = jnp.transpose(x, (1, 0, 2)).reshape(ROWS, D)
    out2d = _run(x2d, idx2d, W, b2d)
    return jnp.transpose(out2d.reshape(F, B, D), (1, 0, 2))
